# Initial kernel scaffold; baseline (speedup 1.0000x reference)
import jax
import jax.numpy as jnp
from jax.experimental import pallas as pl


def kernel(x, emb_weight):
    raise NotImplementedError("write your pallas kernel here")


# speedup vs baseline: 5.1162x; 5.1162x over previous
import functools

import jax
import jax.numpy as jnp
from jax import lax
from jax.experimental import pallas as pl
from jax.experimental.pallas import tpu as pltpu
from jax.experimental.pallas import tpu_sc as plsc

VOCAB = 1000
DIM = 256
CHUNK = 128
IDX_BLOCK = 16


@functools.cache
def _build(B):
    info = plsc.get_sparse_core_info()
    NC, NS = info.num_cores, info.num_subcores
    NW = NC * NS
    b_per_w = B // NW
    assert b_per_w * NW == B and b_per_w % (CHUNK * IDX_BLOCK) == 0
    n_blocks = b_per_w // (CHUNK * IDX_BLOCK)
    mesh = plsc.VectorSubcoreMesh(core_axis_name="c", subcore_axis_name="s")

    @functools.partial(
        pl.kernel,
        mesh=mesh,
        out_type=jax.ShapeDtypeStruct((B, DIM), jnp.float32),
        scratch_types=[
            pltpu.VMEM((IDX_BLOCK, CHUNK), jnp.int32),
            pltpu.VMEM((2, CHUNK, DIM), jnp.float32),
            pltpu.SemaphoreType.DMA,
            pltpu.SemaphoreType.DMA,
            pltpu.SemaphoreType.DMA,
            pltpu.SemaphoreType.DMA,
        ],
    )
    def lookup(table_hbm, idx_hbm, out_hbm, idx_v, rows_v, g0, g1, w0, w1):
        wid = lax.axis_index("s") * NC + lax.axis_index("c")
        base = wid * b_per_w
        gsem = (g0, g1)
        wsem = (w0, w1)

        def fire_gather(j, buf):
            pltpu.async_copy(table_hbm.at[idx_v.at[j]], rows_v.at[buf],
                             gsem[buf])

        def wait_gather(buf):
            pltpu.make_async_copy(out_hbm.at[pl.ds(0, CHUNK)],
                                  rows_v.at[buf], gsem[buf]).wait()

        def fire_write(pos, buf):
            pltpu.async_copy(rows_v.at[buf], out_hbm.at[pl.ds(pos, CHUNK)],
                             wsem[buf])

        def wait_write(buf):
            pltpu.make_async_copy(rows_v.at[buf],
                                  out_hbm.at[pl.ds(0, CHUNK)],
                                  wsem[buf]).wait()

        pltpu.sync_copy(idx_hbm.at[wid, 0], idx_v)
        for j in range(IDX_BLOCK):
            buf = j & 1
            if j >= 2:
                wait_write(buf)
            fire_gather(j, buf)
            if j >= 1:
                wait_gather(1 - buf)
                fire_write(base + (j - 1) * CHUNK, 1 - buf)

        def block(ob, carry):
            pltpu.sync_copy(idx_hbm.at[wid, ob], idx_v)
            g0_pos = base + ob * IDX_BLOCK * CHUNK
            for j in range(IDX_BLOCK):
                buf = j & 1
                wait_write(buf)
                fire_gather(j, buf)
                wait_gather(1 - buf)
                fire_write(g0_pos + (j - 1) * CHUNK, 1 - buf)
            return carry

        lax.fori_loop(1, n_blocks, block, 0, unroll=False)

        last = n_blocks * IDX_BLOCK - 1
        buf = last & 1
        wait_gather(buf)
        fire_write(base + last * CHUNK, buf)
        wait_write(1 - buf)
        wait_write(buf)

    def run(table, idx_flat):
        idx4 = idx_flat.reshape(NW, n_blocks, IDX_BLOCK, CHUNK)
        return lookup(table, idx4)

    return run


def kernel(x, emb_weight):
    b, s = x.shape
    idx = x.reshape(-1).astype(jnp.int32)
    out = _build(idx.shape[0])(emb_weight, idx)
    return out.reshape(b, s, DIM)


# speedup vs baseline: 5.1264x; 1.0020x over previous
import functools

import jax
import jax.numpy as jnp
from jax import lax
from jax.experimental import pallas as pl
from jax.experimental.pallas import tpu as pltpu
from jax.experimental.pallas import tpu_sc as plsc

VOCAB = 1000
DIM = 256
CHUNK = 128
IDX_BLOCK = 32
NBUF = 3


@functools.cache
def _build(B):
    info = plsc.get_sparse_core_info()
    NC, NS = info.num_cores, info.num_subcores
    NW = NC * NS
    b_per_w = B // NW
    assert b_per_w * NW == B and b_per_w % (CHUNK * IDX_BLOCK) == 0
    n_blocks = b_per_w // (CHUNK * IDX_BLOCK)
    n_chunks = b_per_w // CHUNK
    assert n_blocks >= 3
    mesh = plsc.VectorSubcoreMesh(core_axis_name="c", subcore_axis_name="s")

    @functools.partial(
        pl.kernel,
        mesh=mesh,
        out_type=jax.ShapeDtypeStruct((B, DIM), jnp.float32),
        scratch_types=[
            pltpu.VMEM((3, IDX_BLOCK, CHUNK), jnp.int32),
            pltpu.VMEM((NBUF, CHUNK, DIM), jnp.float32),
            pltpu.SemaphoreType.DMA((3,)),
            pltpu.SemaphoreType.DMA((NBUF,)),
            pltpu.SemaphoreType.DMA((NBUF,)),
        ],
    )
    def lookup(table_hbm, idx_hbm, out_hbm, idx_v, rows_v, isem, gsem, wsem):
        wid = lax.axis_index("s") * NC + lax.axis_index("c")
        base = wid * b_per_w

        def stage(ob, slot):
            pltpu.async_copy(idx_hbm.at[wid, ob], idx_v.at[slot],
                             isem.at[slot])

        def wait_idx(slot):
            pltpu.make_async_copy(idx_hbm.at[wid, 0], idx_v.at[slot],
                                  isem.at[slot]).wait()

        def fire_gather(slot, j, buf):
            pltpu.async_copy(table_hbm.at[idx_v.at[slot, j]],
                             rows_v.at[buf], gsem.at[buf])

        def wait_gather(buf):
            pltpu.make_async_copy(out_hbm.at[pl.ds(0, CHUNK)],
                                  rows_v.at[buf], gsem.at[buf]).wait()

        def fire_write(pos, buf):
            pltpu.async_copy(rows_v.at[buf], out_hbm.at[pl.ds(pos, CHUNK)],
                             wsem.at[buf])

        def wait_write(buf):
            pltpu.make_async_copy(rows_v.at[buf],
                                  out_hbm.at[pl.ds(0, CHUNK)],
                                  wsem.at[buf]).wait()

        stage(0, 0)
        stage(1, 1)
        stage(2, 2)
        wait_idx(0)
        for j in range(IDX_BLOCK):
            buf = j % NBUF
            if j >= NBUF:
                wait_write(buf)
            fire_gather(0, j, buf)
            if j >= 1:
                pb = (j - 1) % NBUF
                wait_gather(pb)
                fire_write(base + (j - 1) * CHUNK, pb)

        def block(ob, carry):
            slot = ob % 3
            wait_idx(slot)
            g0 = ob * IDX_BLOCK
            for j in range(IDX_BLOCK):
                buf = (g0 + j) % NBUF
                wait_write(buf)
                fire_gather(slot, j, buf)
                pb = (g0 + j - 1) % NBUF
                wait_gather(pb)
                fire_write(base + (g0 + j - 1) * CHUNK, pb)

            @pl.when(ob + 2 < n_blocks)
            def _():
                stage(ob + 2, (ob + 2) % 3)

            return carry

        lax.fori_loop(1, n_blocks, block, 0, unroll=False)

        last = n_chunks - 1
        lb = last % NBUF
        wait_gather(lb)
        fire_write(base + last * CHUNK, lb)
        for k in range(NBUF):
            wait_write((last - k) % NBUF)

    def run(table, idx_flat):
        idx4 = idx_flat.reshape(NW, n_blocks, IDX_BLOCK, CHUNK)
        return lookup(table, idx4)

    return run


def kernel(x, emb_weight):
    b, s = x.shape
    idx = x.reshape(-1).astype(jnp.int32)
    out = _build(idx.shape[0])(emb_weight, idx)
    return out.reshape(b, s, DIM)
